# Initial kernel scaffold; baseline (speedup 1.0000x reference)
import jax
import jax.numpy as jnp
from jax.experimental import pallas as pl


def kernel(x, edge_index, W1_l, b1_l, W1_r, W2_l, b2_l, W2_r, Wc, bc):
    raise NotImplementedError("write your pallas kernel here")


# speedup vs baseline: 7.4664x; 7.4664x over previous
import functools

import jax
import jax.numpy as jnp
from jax import lax
from jax.experimental import pallas as pl
from jax.experimental.pallas import tpu as pltpu
from jax.experimental.pallas import tpu_sc as plsc

N_NODES = 10000
N_EDGES = 320000
D = 128
NC = 2
NS = 16
NW = NC * NS
EW = N_EDGES // NW
CH = 80
NCHUNK = EW // CH
SCH = 25
NSLAB = NCHUNK // SCH
N_PAD = 10240
ROWS_PER_TILE = N_PAD // NS

@functools.lru_cache(maxsize=None)
def _make_sc_agg(with_counts: bool):
    out_type = [jax.ShapeDtypeStruct((NC, N_PAD, D), jnp.float32)]
    scratch = [
        pltpu.VMEM((SCH, CH), jnp.int32),
        pltpu.VMEM((SCH, CH), jnp.int32),
        pltpu.VMEM((CH, D), jnp.float32),
        pltpu.VMEM_SHARED((N_PAD, D), jnp.float32),
        pltpu.SemaphoreType.DMA,
    ]
    if with_counts:
        out_type.append(jax.ShapeDtypeStruct((NW, N_PAD), jnp.float32))
        scratch.insert(3, pltpu.VMEM((N_PAD,), jnp.float32))

    def body(feat, src3, dst3, *rest):
        if with_counts:
            aggp, cntp, src_v, dst_v, rows_v, cnt_v, agg_sh, sem = rest
        else:
            aggp, src_v, dst_v, rows_v, agg_sh, sem = rest
        c = lax.axis_index("c")
        s = lax.axis_index("s")
        wid = s * NC + c

        zeros16 = jnp.zeros((16,), jnp.float32)

        def zero_row(r, carry):
            for q in range(D // 16):
                rows_v[r, pl.ds(q * 16, 16)] = zeros16
            return carry

        lax.fori_loop(0, CH, zero_row, 0)

        base = s * ROWS_PER_TILE
        nfull = ROWS_PER_TILE // CH
        for k in range(nfull):
            pltpu.sync_copy(rows_v, agg_sh.at[pl.ds(base + k * CH, CH)])

        if with_counts:
            def zero_cnt(i, carry):
                cnt_v[pl.ds(i * 16, 16)] = zeros16
                return carry
            lax.fori_loop(0, N_PAD // 16, zero_cnt, 0)

        plsc.subcore_barrier()

        ones16 = jnp.ones((16,), jnp.float32)

        def slab(g, carry):
            pltpu.sync_copy(src3.at[wid, g], src_v)
            pltpu.sync_copy(dst3.at[wid, g], dst_v)

            def chunk(j, carry2):
                pltpu.async_copy(feat.at[src_v.at[j]], rows_v, sem).wait()
                pltpu.sync_copy(rows_v, agg_sh.at[dst_v.at[j]], add=True)
                if with_counts:
                    for q in range(CH // 16):
                        idx = dst_v[j, pl.ds(q * 16, 16)]
                        plsc.addupdate_scatter(cnt_v, [idx], ones16)
                return carry2

            lax.fori_loop(0, SCH, chunk, 0)
            return carry

        lax.fori_loop(0, NSLAB, slab, 0)

        plsc.subcore_barrier()
        pltpu.sync_copy(agg_sh.at[pl.ds(base, ROWS_PER_TILE)],
                        aggp.at[c, pl.ds(base, ROWS_PER_TILE)])
        if with_counts:
            pltpu.sync_copy(cnt_v, cntp.at[wid])

    mesh = plsc.VectorSubcoreMesh(core_axis_name="c", subcore_axis_name="s",
                                  num_cores=NC, num_subcores=NS)
    return pl.kernel(
        body, out_type=out_type, mesh=mesh, scratch_types=scratch,
        compiler_params=pltpu.CompilerParams(needs_layout_passes=False))


_R = 1280
_GRID = N_PAD // _R
_HI = jax.lax.Precision.HIGHEST


def _dense1_body(aggp, cntp, x, wlT, bl, wrT, h1, inv):
    agg = aggp[0] + aggp[1]
    cnt = cntp[...]
    ones = jnp.ones((NW, 1), jnp.float32)
    total = lax.dot_general(cnt, ones, (((0,), (0,)), ((), ())),
                            preferred_element_type=jnp.float32,
                            precision=_HI)
    iv = 1.0 / jnp.maximum(total, 1.0)
    mean = agg * iv
    h = (jnp.dot(mean, wlT[...], precision=_HI,
                 preferred_element_type=jnp.float32)
         + bl[...]
         + jnp.dot(x[...], wrT[...], precision=_HI,
                   preferred_element_type=jnp.float32))
    h1[...] = jnp.maximum(h, 0.0)
    inv[...] = iv


def _dense2_body(aggp, inv, h1, wlT, bl, wrT, wcT, bc, out):
    agg = aggp[0] + aggp[1]
    mean = agg * inv[...]
    h = (jnp.dot(mean, wlT[...], precision=_HI,
                 preferred_element_type=jnp.float32)
         + bl[...]
         + jnp.dot(h1[...], wrT[...], precision=_HI,
                   preferred_element_type=jnp.float32))
    out[...] = jnp.dot(h, wcT[...], precision=_HI,
                       preferred_element_type=jnp.float32) + bc[...]


_dense1 = pl.pallas_call(
    _dense1_body,
    grid=(_GRID,),
    in_specs=[
        pl.BlockSpec((NC, _R, D), lambda i: (0, i, 0)),
        pl.BlockSpec((NW, _R), lambda i: (0, i)),
        pl.BlockSpec((_R, D), lambda i: (i, 0)),
        pl.BlockSpec((D, D), lambda i: (0, 0)),
        pl.BlockSpec((1, D), lambda i: (0, 0)),
        pl.BlockSpec((D, D), lambda i: (0, 0)),
    ],
    out_specs=[
        pl.BlockSpec((_R, D), lambda i: (i, 0)),
        pl.BlockSpec((_R, 1), lambda i: (i, 0)),
    ],
    out_shape=[
        jax.ShapeDtypeStruct((N_PAD, D), jnp.float32),
        jax.ShapeDtypeStruct((N_PAD, 1), jnp.float32),
    ],
)

_dense2 = pl.pallas_call(
    _dense2_body,
    grid=(_GRID,),
    in_specs=[
        pl.BlockSpec((NC, _R, D), lambda i: (0, i, 0)),
        pl.BlockSpec((_R, 1), lambda i: (i, 0)),
        pl.BlockSpec((_R, D), lambda i: (i, 0)),
        pl.BlockSpec((D, D), lambda i: (0, 0)),
        pl.BlockSpec((1, D), lambda i: (0, 0)),
        pl.BlockSpec((D, D), lambda i: (0, 0)),
        pl.BlockSpec((D, 2), lambda i: (0, 0)),
        pl.BlockSpec((1, 2), lambda i: (0, 0)),
    ],
    out_specs=pl.BlockSpec((_R, 2), lambda i: (i, 0)),
    out_shape=jax.ShapeDtypeStruct((N_PAD, 2), jnp.float32),
)


def kernel(x, edge_index, W1_l, b1_l, W1_r, W2_l, b2_l, W2_r, Wc, bc):
    src3 = edge_index[0].astype(jnp.int32).reshape(NW, NSLAB, SCH, CH)
    dst3 = edge_index[1].astype(jnp.int32).reshape(NW, NSLAB, SCH, CH)
    xp = jnp.pad(x, ((0, N_PAD - N_NODES), (0, 0)))

    aggp1, cntp = _make_sc_agg(True)(x, src3, dst3)
    h1, inv = _dense1(aggp1, cntp, xp, W1_l.T, b1_l.reshape(1, D),
                      W1_r.T)
    (aggp2,) = _make_sc_agg(False)(h1, src3, dst3)
    out = _dense2(aggp2, inv, h1, W2_l.T, b2_l.reshape(1, D), W2_r.T,
                  Wc.T, bc.reshape(1, 2))
    return out[:N_NODES]


# speedup vs baseline: 9.2900x; 1.2442x over previous
import functools

import jax
import jax.numpy as jnp
from jax import lax
from jax.experimental import pallas as pl
from jax.experimental.pallas import tpu as pltpu
from jax.experimental.pallas import tpu_sc as plsc

N_NODES = 10000
N_EDGES = 320000
D = 128
NC = 2
NS = 16
NW = NC * NS
EW = N_EDGES // NW
CH = 80
NCHUNK = EW // CH
SCH = 25
NSLAB = NCHUNK // SCH
N_PAD = 10240
ROWS_PER_TILE = N_PAD // NS

@functools.lru_cache(maxsize=None)
def _make_sc_agg(with_counts: bool):
    out_type = [jax.ShapeDtypeStruct((NC, N_PAD, D), jnp.float32)]
    scratch = [
        pltpu.VMEM((SCH, CH), jnp.int32),
        pltpu.VMEM((SCH, CH), jnp.int32),
        pltpu.VMEM((CH, D), jnp.float32),
        pltpu.VMEM((CH, D), jnp.float32),
        pltpu.VMEM_SHARED((N_PAD, D), jnp.float32),
        pltpu.SemaphoreType.DMA,
        pltpu.SemaphoreType.DMA,
    ]
    if with_counts:
        out_type.append(jax.ShapeDtypeStruct((NW, N_PAD), jnp.float32))
        scratch.insert(4, pltpu.VMEM((N_PAD,), jnp.float32))

    def body(feat, src3, dst3, *rest):
        if with_counts:
            (aggp, cntp, src_v, dst_v, rows0, rows1, cnt_v, agg_sh,
             sem0, sem1) = rest
        else:
            aggp, src_v, dst_v, rows0, rows1, agg_sh, sem0, sem1 = rest
        bufs = (rows0, rows1)
        sems = (sem0, sem1)
        c = lax.axis_index("c")
        s = lax.axis_index("s")
        wid = s * NC + c

        zeros16 = jnp.zeros((16,), jnp.float32)

        def zero_row(r, carry):
            for q in range(D // 16):
                rows0[r, pl.ds(q * 16, 16)] = zeros16
            return carry

        lax.fori_loop(0, CH, zero_row, 0)

        base = s * ROWS_PER_TILE
        nfull = ROWS_PER_TILE // CH
        for k in range(nfull):
            pltpu.sync_copy(rows0, agg_sh.at[pl.ds(base + k * CH, CH)])

        if with_counts:
            def zero_cnt(i, carry):
                cnt_v[pl.ds(i * 16, 16)] = zeros16
                return carry
            lax.fori_loop(0, N_PAD // 16, zero_cnt, 0)

        plsc.subcore_barrier()

        ones16 = jnp.ones((16,), jnp.float32)

        def count_chunk(k):
            for q in range(CH // 16):
                idx = dst_v[k, pl.ds(q * 16, 16)]
                plsc.addupdate_scatter(cnt_v, [idx], ones16)

        def slab(g, carry):
            pltpu.sync_copy(src3.at[wid, g], src_v)
            pltpu.sync_copy(dst3.at[wid, g], dst_v)

            pltpu.async_copy(feat.at[src_v.at[0]], rows0, sem0)

            def pair(j, carry2):
                for b in range(2):
                    k = 2 * j + b
                    pltpu.make_async_copy(
                        feat.at[src_v.at[k]], bufs[b], sems[b]).wait()
                    pltpu.async_copy(
                        feat.at[src_v.at[k + 1]], bufs[1 - b], sems[1 - b])
                    pltpu.sync_copy(bufs[b], agg_sh.at[dst_v.at[k]], add=True)
                    if with_counts:
                        count_chunk(k)
                return carry2

            lax.fori_loop(0, (SCH - 1) // 2, pair, 0)
            last = SCH - 1
            pltpu.make_async_copy(feat.at[src_v.at[last]], rows0, sem0).wait()
            pltpu.sync_copy(rows0, agg_sh.at[dst_v.at[last]], add=True)
            if with_counts:
                count_chunk(last)
            return carry

        lax.fori_loop(0, NSLAB, slab, 0)

        plsc.subcore_barrier()
        pltpu.sync_copy(agg_sh.at[pl.ds(base, ROWS_PER_TILE)],
                        aggp.at[c, pl.ds(base, ROWS_PER_TILE)])
        if with_counts:
            pltpu.sync_copy(cnt_v, cntp.at[wid])

    mesh = plsc.VectorSubcoreMesh(core_axis_name="c", subcore_axis_name="s",
                                  num_cores=NC, num_subcores=NS)
    return pl.kernel(
        body, out_type=out_type, mesh=mesh, scratch_types=scratch,
        compiler_params=pltpu.CompilerParams(needs_layout_passes=False))


_R = 1280
_GRID = N_PAD // _R
_HI = jax.lax.Precision.HIGHEST


def _dense1_body(aggp, cntp, x, wlT, bl, wrT, h1, inv):
    agg = aggp[0] + aggp[1]
    cnt = cntp[...]
    ones = jnp.ones((NW, 1), jnp.float32)
    total = lax.dot_general(cnt, ones, (((0,), (0,)), ((), ())),
                            preferred_element_type=jnp.float32,
                            precision=_HI)
    iv = 1.0 / jnp.maximum(total, 1.0)
    mean = agg * iv
    h = (jnp.dot(mean, wlT[...], precision=_HI,
                 preferred_element_type=jnp.float32)
         + bl[...]
         + jnp.dot(x[...], wrT[...], precision=_HI,
                   preferred_element_type=jnp.float32))
    h1[...] = jnp.maximum(h, 0.0)
    inv[...] = iv


def _dense2_body(aggp, inv, h1, wlT, bl, wrT, wcT, bc, out):
    agg = aggp[0] + aggp[1]
    mean = agg * inv[...]
    h = (jnp.dot(mean, wlT[...], precision=_HI,
                 preferred_element_type=jnp.float32)
         + bl[...]
         + jnp.dot(h1[...], wrT[...], precision=_HI,
                   preferred_element_type=jnp.float32))
    out[...] = jnp.dot(h, wcT[...], precision=_HI,
                       preferred_element_type=jnp.float32) + bc[...]


_dense1 = pl.pallas_call(
    _dense1_body,
    grid=(_GRID,),
    in_specs=[
        pl.BlockSpec((NC, _R, D), lambda i: (0, i, 0)),
        pl.BlockSpec((NW, _R), lambda i: (0, i)),
        pl.BlockSpec((_R, D), lambda i: (i, 0)),
        pl.BlockSpec((D, D), lambda i: (0, 0)),
        pl.BlockSpec((1, D), lambda i: (0, 0)),
        pl.BlockSpec((D, D), lambda i: (0, 0)),
    ],
    out_specs=[
        pl.BlockSpec((_R, D), lambda i: (i, 0)),
        pl.BlockSpec((_R, 1), lambda i: (i, 0)),
    ],
    out_shape=[
        jax.ShapeDtypeStruct((N_PAD, D), jnp.float32),
        jax.ShapeDtypeStruct((N_PAD, 1), jnp.float32),
    ],
)

_dense2 = pl.pallas_call(
    _dense2_body,
    grid=(_GRID,),
    in_specs=[
        pl.BlockSpec((NC, _R, D), lambda i: (0, i, 0)),
        pl.BlockSpec((_R, 1), lambda i: (i, 0)),
        pl.BlockSpec((_R, D), lambda i: (i, 0)),
        pl.BlockSpec((D, D), lambda i: (0, 0)),
        pl.BlockSpec((1, D), lambda i: (0, 0)),
        pl.BlockSpec((D, D), lambda i: (0, 0)),
        pl.BlockSpec((D, 2), lambda i: (0, 0)),
        pl.BlockSpec((1, 2), lambda i: (0, 0)),
    ],
    out_specs=pl.BlockSpec((_R, 2), lambda i: (i, 0)),
    out_shape=jax.ShapeDtypeStruct((N_PAD, 2), jnp.float32),
)


def kernel(x, edge_index, W1_l, b1_l, W1_r, W2_l, b2_l, W2_r, Wc, bc):
    src3 = edge_index[0].astype(jnp.int32).reshape(NW, NSLAB, SCH, CH)
    dst3 = edge_index[1].astype(jnp.int32).reshape(NW, NSLAB, SCH, CH)
    xp = jnp.pad(x, ((0, N_PAD - N_NODES), (0, 0)))

    aggp1, cntp = _make_sc_agg(True)(x, src3, dst3)
    h1, inv = _dense1(aggp1, cntp, xp, W1_l.T, b1_l.reshape(1, D),
                      W1_r.T)
    (aggp2,) = _make_sc_agg(False)(h1, src3, dst3)
    out = _dense2(aggp2, inv, h1, W2_l.T, b2_l.reshape(1, D), W2_r.T,
                  Wc.T, bc.reshape(1, 2))
    return out[:N_NODES]


# speedup vs baseline: 9.3536x; 1.0068x over previous
import functools

import jax
import jax.numpy as jnp
from jax import lax
from jax.experimental import pallas as pl
from jax.experimental.pallas import tpu as pltpu
from jax.experimental.pallas import tpu_sc as plsc

N_NODES = 10000
N_EDGES = 320000
D = 128
NC = 2
NS = 16
NW = NC * NS
EW = N_EDGES // NW
CH = 80
NCHUNK = EW // CH
SCH = 25
NSLAB = NCHUNK // SCH
N_PAD = 10240
ROWS_PER_TILE = N_PAD // NS

@functools.lru_cache(maxsize=None)
def _make_sc_agg(with_counts: bool):
    out_type = [jax.ShapeDtypeStruct((NC, N_PAD, D), jnp.float32)]
    scratch = [
        pltpu.VMEM((SCH, CH), jnp.int32),
        pltpu.VMEM((SCH, CH), jnp.int32),
        pltpu.VMEM((CH, D), jnp.float32),
        pltpu.VMEM((CH, D), jnp.float32),
        pltpu.VMEM_SHARED((N_PAD, D), jnp.float32),
        pltpu.SemaphoreType.DMA,
        pltpu.SemaphoreType.DMA,
    ]
    if with_counts:
        out_type.append(jax.ShapeDtypeStruct((NW, N_PAD), jnp.float32))
        scratch.insert(4, pltpu.VMEM((N_PAD,), jnp.float32))

    def body(feat, src3, dst3, *rest):
        if with_counts:
            (aggp, cntp, src_v, dst_v, rows0, rows1, cnt_v, agg_sh,
             sem0, sem1) = rest
        else:
            aggp, src_v, dst_v, rows0, rows1, agg_sh, sem0, sem1 = rest
        bufs = (rows0, rows1)
        sems = (sem0, sem1)
        c = lax.axis_index("c")
        s = lax.axis_index("s")
        wid = s * NC + c

        zeros16 = jnp.zeros((16,), jnp.float32)

        def zero_row(r, carry):
            for q in range(D // 16):
                rows0[r, pl.ds(q * 16, 16)] = zeros16
            return carry

        lax.fori_loop(0, CH, zero_row, 0)

        base = s * ROWS_PER_TILE
        nfull = ROWS_PER_TILE // CH
        for k in range(nfull):
            pltpu.sync_copy(rows0, agg_sh.at[pl.ds(base + k * CH, CH)])

        if with_counts:
            def zero_cnt(i, carry):
                cnt_v[pl.ds(i * 16, 16)] = zeros16
                return carry
            lax.fori_loop(0, N_PAD // 16, zero_cnt, 0)

        plsc.subcore_barrier()

        ones16 = jnp.ones((16,), jnp.float32)

        def count_chunk(k):
            for q in range(CH // 16):
                idx = dst_v[k, pl.ds(q * 16, 16)]
                plsc.addupdate_scatter(cnt_v, [idx], ones16)

        def slab(g, carry):
            pltpu.sync_copy(src3.at[wid, g], src_v)
            pltpu.sync_copy(dst3.at[wid, g], dst_v)

            pltpu.async_copy(feat.at[src_v.at[0]], rows0, sem0)

            def pair(j, carry2):
                for b in range(2):
                    k = 2 * j + b
                    pltpu.make_async_copy(
                        feat.at[src_v.at[k]], bufs[b], sems[b]).wait()
                    pltpu.async_copy(
                        feat.at[src_v.at[k + 1]], bufs[1 - b], sems[1 - b])
                    pltpu.sync_copy(bufs[b], agg_sh.at[dst_v.at[k]], add=True)
                    if with_counts:
                        count_chunk(k)
                return carry2

            lax.fori_loop(0, (SCH - 1) // 2, pair, 0)
            last = SCH - 1
            pltpu.make_async_copy(feat.at[src_v.at[last]], rows0, sem0).wait()
            pltpu.sync_copy(rows0, agg_sh.at[dst_v.at[last]], add=True)
            if with_counts:
                count_chunk(last)
            return carry

        lax.fori_loop(0, NSLAB, slab, 0)

        plsc.subcore_barrier()
        pltpu.sync_copy(agg_sh.at[pl.ds(base, ROWS_PER_TILE)],
                        aggp.at[c, pl.ds(base, ROWS_PER_TILE)])
        if with_counts:
            pltpu.sync_copy(cnt_v, cntp.at[wid])

    mesh = plsc.VectorSubcoreMesh(core_axis_name="c", subcore_axis_name="s",
                                  num_cores=NC, num_subcores=NS)
    return pl.kernel(
        body, out_type=out_type, mesh=mesh, scratch_types=scratch,
        compiler_params=pltpu.CompilerParams(needs_layout_passes=False))


_R = 1280
_GRID = N_PAD // _R
_HI = jax.lax.Precision.HIGHEST


def _dot_t(a, w):
    return lax.dot_general(a, w, (((1,), (1,)), ((), ())),
                           preferred_element_type=jnp.float32,
                           precision=_HI)


def _dense1_body(aggp, cntp, x, wl, bl, wr, h1, inv):
    agg = aggp[0] + aggp[1]
    cnt = cntp[...]
    ones = jnp.ones((NW, 1), jnp.float32)
    total = lax.dot_general(cnt, ones, (((0,), (0,)), ((), ())),
                            preferred_element_type=jnp.float32,
                            precision=_HI)
    iv = 1.0 / jnp.maximum(total, 1.0)
    mean = agg * iv
    h = _dot_t(mean, wl[...]) + bl[...] + _dot_t(x[...], wr[...])
    h1[...] = jnp.maximum(h, 0.0)
    inv[...] = iv


def _dense2_body(aggp, inv, h1, wl, bl, wr, wc, bc, out):
    agg = aggp[0] + aggp[1]
    mean = agg * inv[...]
    h = _dot_t(mean, wl[...]) + bl[...] + _dot_t(h1[...], wr[...])
    out[...] = _dot_t(h, wc[...]) + bc[...]


_dense1 = pl.pallas_call(
    _dense1_body,
    grid=(_GRID,),
    in_specs=[
        pl.BlockSpec((NC, _R, D), lambda i: (0, i, 0)),
        pl.BlockSpec((NW, _R), lambda i: (0, i)),
        pl.BlockSpec((_R, D), lambda i: (i, 0)),
        pl.BlockSpec((D, D), lambda i: (0, 0)),
        pl.BlockSpec((1, D), lambda i: (0, 0)),
        pl.BlockSpec((D, D), lambda i: (0, 0)),
    ],
    out_specs=[
        pl.BlockSpec((_R, D), lambda i: (i, 0)),
        pl.BlockSpec((_R, 1), lambda i: (i, 0)),
    ],
    out_shape=[
        jax.ShapeDtypeStruct((N_PAD, D), jnp.float32),
        jax.ShapeDtypeStruct((N_PAD, 1), jnp.float32),
    ],
)

_dense2 = pl.pallas_call(
    _dense2_body,
    grid=(_GRID,),
    in_specs=[
        pl.BlockSpec((NC, _R, D), lambda i: (0, i, 0)),
        pl.BlockSpec((_R, 1), lambda i: (i, 0)),
        pl.BlockSpec((_R, D), lambda i: (i, 0)),
        pl.BlockSpec((D, D), lambda i: (0, 0)),
        pl.BlockSpec((1, D), lambda i: (0, 0)),
        pl.BlockSpec((D, D), lambda i: (0, 0)),
        pl.BlockSpec((2, D), lambda i: (0, 0)),
        pl.BlockSpec((1, 2), lambda i: (0, 0)),
    ],
    out_specs=pl.BlockSpec((_R, 2), lambda i: (i, 0)),
    out_shape=jax.ShapeDtypeStruct((N_NODES, 2), jnp.float32),
)


def kernel(x, edge_index, W1_l, b1_l, W1_r, W2_l, b2_l, W2_r, Wc, bc):
    src3 = edge_index[0].astype(jnp.int32).reshape(NW, NSLAB, SCH, CH)
    dst3 = edge_index[1].astype(jnp.int32).reshape(NW, NSLAB, SCH, CH)

    aggp1, cntp = _make_sc_agg(True)(x, src3, dst3)
    h1, inv = _dense1(aggp1, cntp, x, W1_l, b1_l.reshape(1, D), W1_r)
    (aggp2,) = _make_sc_agg(False)(h1, src3, dst3)
    out = _dense2(aggp2, inv, h1, W2_l, b2_l.reshape(1, D), W2_r,
                  Wc, bc.reshape(1, 2))
    return out


# speedup vs baseline: 12.2067x; 1.3050x over previous
import functools

import jax
import jax.numpy as jnp
from jax import lax
from jax.experimental import pallas as pl
from jax.experimental.pallas import tpu as pltpu
from jax.experimental.pallas import tpu_sc as plsc

N_NODES = 10000
N_EDGES = 320000
D = 128
NC = 2
NS = 16
NW = NC * NS
EW = N_EDGES // NW
CH = 80
NCHUNK = EW // CH
SCH = 25
NSLAB = NCHUNK // SCH
N_PAD = 10240
ROWS_PER_TILE = N_PAD // NS

def _sc_mesh():
    return plsc.VectorSubcoreMesh(core_axis_name="c", subcore_axis_name="s",
                                  num_cores=NC, num_subcores=NS)


@functools.lru_cache(maxsize=None)
def _make_sc_agg():
    out_type = [jax.ShapeDtypeStruct((NC, N_PAD, D), jnp.float32)]
    scratch = [
        pltpu.VMEM((SCH, CH), jnp.int32),
        pltpu.VMEM((SCH, CH), jnp.int32),
        pltpu.VMEM((CH, D), jnp.float32),
        pltpu.VMEM((CH, D), jnp.float32),
        pltpu.VMEM((CH, D), jnp.float32),
        pltpu.VMEM_SHARED((N_PAD, D), jnp.float32),
        pltpu.SemaphoreType.DMA,
        pltpu.SemaphoreType.DMA,
        pltpu.SemaphoreType.DMA,
        pltpu.SemaphoreType.DMA,
        pltpu.SemaphoreType.DMA,
        pltpu.SemaphoreType.DMA,
    ]

    def body(feat, src3, dst3, aggp, src_v, dst_v, r0, r1, r2, agg_sh,
             g0, g1, g2, s0, s1, s2):
        bufs = (r0, r1, r2)
        gsem = (g0, g1, g2)
        ssem = (s0, s1, s2)
        c = lax.axis_index("c")
        s = lax.axis_index("s")
        wid = s * NC + c

        zeros16 = jnp.zeros((16,), jnp.float32)

        def zero_row(r, carry):
            for q in range(D // 16):
                r0[r, pl.ds(q * 16, 16)] = zeros16
            return carry

        lax.fori_loop(0, CH, zero_row, 0)

        base = s * ROWS_PER_TILE
        for k in range(ROWS_PER_TILE // CH):
            pltpu.sync_copy(r0, agg_sh.at[pl.ds(base + k * CH, CH)])

        plsc.subcore_barrier()

        def gather(k, b):
            pltpu.async_copy(feat.at[src_v.at[k]], bufs[b], gsem[b])

        def wait_gather(k, b):
            pltpu.make_async_copy(feat.at[src_v.at[k]], bufs[b],
                                  gsem[b]).wait()

        def scatter(k, b):
            pltpu.async_copy(bufs[b], agg_sh.at[dst_v.at[k]], ssem[b],
                             add=True)

        def wait_scatter(k, b):
            pltpu.make_async_copy(bufs[b], agg_sh.at[dst_v.at[k]],
                                  ssem[b]).wait()

        def slab(g, carry):
            pltpu.sync_copy(src3.at[wid, g], src_v)
            pltpu.sync_copy(dst3.at[wid, g], dst_v)

            gather(0, 0)
            gather(1, 1)

            def tri(j, carry2):
                for b in range(3):
                    k = 3 * j + b
                    wait_gather(k, b)
                    @pl.when(k + 2 <= SCH - 1)
                    def _():
                        gather(k + 2, (b + 2) % 3)
                    scatter(k, b)
                    wait_scatter(k, b)
                return carry2

            lax.fori_loop(0, (SCH - 1) // 3, tri, 0)
            last = SCH - 1
            bl_ = last % 3
            wait_gather(last, bl_)
            scatter(last, bl_)
            wait_scatter(last, bl_)
            return carry

        lax.fori_loop(0, NSLAB, slab, 0)

        plsc.subcore_barrier()
        pltpu.sync_copy(agg_sh.at[pl.ds(base, ROWS_PER_TILE)],
                        aggp.at[c, pl.ds(base, ROWS_PER_TILE)])

    return pl.kernel(
        body, out_type=out_type, mesh=_sc_mesh(), scratch_types=scratch,
        compiler_params=pltpu.CompilerParams(needs_layout_passes=False))


@functools.lru_cache(maxsize=None)
def _make_sc_counts():
    def body(dst3, cntp, dst_v, cnt_v):
        c = lax.axis_index("c")
        s = lax.axis_index("s")
        wid = s * NC + c

        zeros16 = jnp.zeros((16,), jnp.float32)

        def zero_cnt(i, carry):
            cnt_v[pl.ds(i * 16, 16)] = zeros16
            return carry

        lax.fori_loop(0, N_PAD // 16, zero_cnt, 0)

        ones16 = jnp.ones((16,), jnp.float32)

        def slab(g, carry):
            pltpu.sync_copy(dst3.at[wid, g], dst_v)

            def chunk(j, carry2):
                for q in range(CH // 16):
                    idx = dst_v[j, pl.ds(q * 16, 16)]
                    plsc.addupdate_scatter(cnt_v, [idx], ones16)
                return carry2

            lax.fori_loop(0, SCH, chunk, 0)
            return carry

        lax.fori_loop(0, NSLAB, slab, 0)
        pltpu.sync_copy(cnt_v, cntp.at[wid])

    return pl.kernel(
        body,
        out_type=[jax.ShapeDtypeStruct((NW, N_PAD), jnp.float32)],
        mesh=_sc_mesh(),
        scratch_types=[
            pltpu.VMEM((SCH, CH), jnp.int32),
            pltpu.VMEM((N_PAD,), jnp.float32),
        ],
        compiler_params=pltpu.CompilerParams(needs_layout_passes=False))


_R = 1280
_GRID = N_PAD // _R
_HI = jax.lax.Precision.HIGHEST


def _dot_t(a, w):
    return lax.dot_general(a, w, (((1,), (1,)), ((), ())),
                           preferred_element_type=jnp.float32,
                           precision=_HI)


def _dense1_body(aggp, cntp, x, wl, bl, wr, h1, inv):
    agg = aggp[0] + aggp[1]
    cnt = cntp[...]
    ones = jnp.ones((NW, 1), jnp.float32)
    total = lax.dot_general(cnt, ones, (((0,), (0,)), ((), ())),
                            preferred_element_type=jnp.float32,
                            precision=_HI)
    iv = 1.0 / jnp.maximum(total, 1.0)
    mean = agg * iv
    h = _dot_t(mean, wl[...]) + bl[...] + _dot_t(x[...], wr[...])
    h1[...] = jnp.maximum(h, 0.0)
    inv[...] = iv


def _dense2_body(aggp, inv, h1, wl, bl, wr, wc, bc, out):
    agg = aggp[0] + aggp[1]
    mean = agg * inv[...]
    h = _dot_t(mean, wl[...]) + bl[...] + _dot_t(h1[...], wr[...])
    out[...] = _dot_t(h, wc[...]) + bc[...]


_dense1 = pl.pallas_call(
    _dense1_body,
    grid=(_GRID,),
    in_specs=[
        pl.BlockSpec((NC, _R, D), lambda i: (0, i, 0)),
        pl.BlockSpec((NW, _R), lambda i: (0, i)),
        pl.BlockSpec((_R, D), lambda i: (i, 0)),
        pl.BlockSpec((D, D), lambda i: (0, 0)),
        pl.BlockSpec((1, D), lambda i: (0, 0)),
        pl.BlockSpec((D, D), lambda i: (0, 0)),
    ],
    out_specs=[
        pl.BlockSpec((_R, D), lambda i: (i, 0)),
        pl.BlockSpec((_R, 1), lambda i: (i, 0)),
    ],
    out_shape=[
        jax.ShapeDtypeStruct((N_PAD, D), jnp.float32),
        jax.ShapeDtypeStruct((N_PAD, 1), jnp.float32),
    ],
)

_dense2 = pl.pallas_call(
    _dense2_body,
    grid=(_GRID,),
    in_specs=[
        pl.BlockSpec((NC, _R, D), lambda i: (0, i, 0)),
        pl.BlockSpec((_R, 1), lambda i: (i, 0)),
        pl.BlockSpec((_R, D), lambda i: (i, 0)),
        pl.BlockSpec((D, D), lambda i: (0, 0)),
        pl.BlockSpec((1, D), lambda i: (0, 0)),
        pl.BlockSpec((D, D), lambda i: (0, 0)),
        pl.BlockSpec((2, D), lambda i: (0, 0)),
        pl.BlockSpec((1, 2), lambda i: (0, 0)),
    ],
    out_specs=pl.BlockSpec((_R, 2), lambda i: (i, 0)),
    out_shape=jax.ShapeDtypeStruct((N_NODES, 2), jnp.float32),
)


def kernel(x, edge_index, W1_l, b1_l, W1_r, W2_l, b2_l, W2_r, Wc, bc):
    src3 = edge_index[0].astype(jnp.int32).reshape(NW, NSLAB, SCH, CH)
    dst3 = edge_index[1].astype(jnp.int32).reshape(NW, NSLAB, SCH, CH)

    (cntp,) = _make_sc_counts()(dst3)
    (aggp1,) = _make_sc_agg()(x, src3, dst3)
    h1, inv = _dense1(aggp1, cntp, x, W1_l, b1_l.reshape(1, D), W1_r)
    (aggp2,) = _make_sc_agg()(h1, src3, dst3)
    out = _dense2(aggp2, inv, h1, W2_l, b2_l.reshape(1, D), W2_r,
                  Wc, bc.reshape(1, 2))
    return out
